# baseline (speedup 1.0000x reference)
import functools

import jax
import jax.numpy as jnp
from jax import lax
from jax.experimental import pallas as pl
from jax.experimental.pallas import tpu as pltpu
from jax.experimental.pallas import tpu_sc as plsc

NC = 2
NS = 16
NW = NC * NS
L = 16

_RSQRT_MAGIC = 0x5F3759DF


def _rsqrt(x):
    i = plsc.bitcast(x, jnp.int32)
    i = _RSQRT_MAGIC - lax.shift_right_logical(i, 1)
    y = plsc.bitcast(i, jnp.float32)
    xh = x * 0.5
    for _ in range(3):
        y = y * (1.5 - xh * y * y)
    return y


def _make_sc_call(B, N, M, K):
    SPB = NW // B
    CW = M // SPB
    NQ = CW // L
    mesh = plsc.VectorSubcoreMesh(core_axis_name="c", subcore_axis_name="s")

    @functools.partial(
        pl.kernel,
        mesh=mesh,
        out_type=jax.ShapeDtypeStruct((NW, CW * K), jnp.float32),
        compiler_params=pltpu.CompilerParams(needs_layout_passes=False),
        scratch_types=[
            pltpu.VMEM((N,), jnp.float32),
            pltpu.VMEM((K * CW,), jnp.int32),
            pltpu.VMEM((K * CW,), jnp.float32),
            pltpu.VMEM((3 * CW,), jnp.float32),
            pltpu.SemaphoreType.DMA,
            pltpu.SemaphoreType.DMA,
            pltpu.SemaphoreType.DMA,
        ],
    )
    def sc_call(tab_hbm, idx_hbm, q_hbm, out_hbm,
                table_v, idx_v, acc_v, q_v, sem0, sem1, sem2):
        c = lax.axis_index("c")
        s = lax.axis_index("s")
        w = c * NS + s
        bidx = w // SPB

        with jax.named_scope("init_dma"):
            cp_i = pltpu.async_copy(idx_hbm.at[w], idx_v, sem0)
            cp_q = pltpu.async_copy(q_hbm.at[w], q_v, sem1)
            cp_t = pltpu.async_copy(tab_hbm.at[bidx * 3], table_v, sem2)
            cp_i.wait()
            cp_q.wait()
            cp_t.wait()

        for comp in range(3):
            if comp:
                with jax.named_scope(f"tab_dma{comp}"):
                    pltpu.sync_copy(tab_hbm.at[bidx * 3 + comp], table_v)

            def jloop(j, carry, comp=comp):
                qv = q_v[pl.ds(comp * CW + j * L, L)]
                for k in range(K):
                    off = k * CW + j * L
                    g = plsc.load_gather(table_v, [idx_v[pl.ds(off, L)]])
                    d = g - qv
                    if comp == 0:
                        acc_v[pl.ds(off, L)] = d * d
                    else:
                        acc_v[pl.ds(off, L)] = acc_v[pl.ds(off, L)] + d * d
                return carry

            with jax.named_scope(f"gather{comp}"):
                lax.fori_loop(0, NQ, jloop, 0)

        def sloop(j, carry):
            base = j * L
            dists = []
            for k in range(K):
                x = acc_v[pl.ds(k * CW + base, L)]
                x = jnp.maximum(x, 1e-30)
                dists.append(x * _rsqrt(x))
            mn = dists[0]
            for k in range(1, K):
                mn = jnp.minimum(mn, dists[k])
            es = []
            tot = None
            for k in range(K):
                e = jnp.exp((mn - dists[k]) * 200.0)
                es.append(e)
                tot = e if tot is None else tot + e
            r = 0.5 / tot
            for k in range(K):
                wv = es[k] * r
                if k == 0:
                    wv = wv + 0.5
                acc_v[pl.ds(k * CW + base, L)] = wv
            return carry

        with jax.named_scope("softmax"):
            lax.fori_loop(0, NQ, sloop, 0)
        with jax.named_scope("out_dma"):
            pltpu.sync_copy(acc_v, out_hbm.at[w])

    return sc_call


def kernel(feature1, xyz1, query_xyz, error, point_index, W, b):
    B, N, _ = xyz1.shape
    M = query_xyz.shape[1]
    K = point_index.shape[2]
    SPB = NW // B
    CW = M // SPB

    tables = xyz1.transpose(0, 2, 1).reshape(B * 3, N)
    idx = (
        point_index.astype(jnp.int32)
        .transpose(0, 2, 1)
        .reshape(B, K, SPB, CW)
        .transpose(0, 2, 1, 3)
        .reshape(NW, K * CW)
    )
    q = (
        query_xyz.transpose(0, 2, 1)
        .reshape(B, 3, SPB, CW)
        .transpose(0, 2, 1, 3)
        .reshape(NW, 3 * CW)
    )

    out = _make_sc_call(B, N, M, K)(tables, idx, q)
    return (
        out.reshape(B, SPB, K, CW)
        .transpose(0, 1, 3, 2)
        .reshape(B, M, K)
    )


# speedup vs baseline: 1.3685x; 1.3685x over previous
import functools

import jax
import jax.numpy as jnp
from jax import lax
from jax.experimental import pallas as pl
from jax.experimental.pallas import tpu as pltpu
from jax.experimental.pallas import tpu_sc as plsc

NC = 2
NS = 16
NW = NC * NS
L = 16

_RSQRT_MAGIC = 0x5F3759DF


def _rsqrt(x):
    i = plsc.bitcast(x, jnp.int32)
    i = _RSQRT_MAGIC - lax.shift_right_logical(i, 1)
    y = plsc.bitcast(i, jnp.float32)
    xh = x * 0.5
    for _ in range(3):
        y = y * (1.5 - xh * y * y)
    return y


def _make_sc_call(B, N, M, K):
    SPB = NW // B
    CW = M // SPB
    NQ = CW // L
    mesh = plsc.VectorSubcoreMesh(core_axis_name="c", subcore_axis_name="s")

    @functools.partial(
        pl.kernel,
        mesh=mesh,
        out_type=jax.ShapeDtypeStruct((NW, CW * K), jnp.float32),
        compiler_params=pltpu.CompilerParams(needs_layout_passes=False),
        scratch_types=[
            pltpu.VMEM((N,), jnp.float32),
            pltpu.VMEM((K * CW,), jnp.int32),
            pltpu.VMEM((K * CW,), jnp.float32),
            pltpu.VMEM((3 * CW,), jnp.float32),
            pltpu.SemaphoreType.DMA,
            pltpu.SemaphoreType.DMA,
            pltpu.SemaphoreType.DMA,
        ],
    )
    def sc_call(tab_hbm, idx_hbm, q_hbm, out_hbm,
                table_v, idx_v, acc_v, q_v, sem0, sem1, sem2):
        c = lax.axis_index("c")
        s = lax.axis_index("s")
        w = c * NS + s
        bidx = w // SPB

        with jax.named_scope("init_dma"):
            cp_i = pltpu.async_copy(idx_hbm.at[w], idx_v, sem0)
            cp_q = pltpu.async_copy(q_hbm.at[w], q_v, sem1)
            cp_t = pltpu.async_copy(tab_hbm.at[bidx * 3], table_v, sem2)
            cp_i.wait()
            cp_q.wait()
            cp_t.wait()

        for comp in range(3):
            if comp:
                with jax.named_scope(f"tab_dma{comp}"):
                    pltpu.sync_copy(tab_hbm.at[bidx * 3 + comp], table_v)

            with jax.named_scope(f"gather{comp}"):
                @plsc.parallel_loop(0, NQ, unroll=4)
                def jloop(j, comp=comp):
                    qv = q_v[pl.ds(comp * CW + j * L, L)]
                    for k in range(K):
                        off = k * CW + j * L
                        g = plsc.load_gather(table_v, [idx_v[pl.ds(off, L)]])
                        d = g - qv
                        if comp == 0:
                            acc_v[pl.ds(off, L)] = d * d
                        else:
                            acc_v[pl.ds(off, L)] = acc_v[pl.ds(off, L)] + d * d

        with jax.named_scope("softmax"):
            @plsc.parallel_loop(0, NQ, unroll=2)
            def sloop(j):
                base = j * L
                dists = []
                for k in range(K):
                    x = acc_v[pl.ds(k * CW + base, L)]
                    x = jnp.maximum(x, 1e-30)
                    dists.append(x * _rsqrt(x))
                mn = dists[0]
                for k in range(1, K):
                    mn = jnp.minimum(mn, dists[k])
                es = []
                tot = None
                for k in range(K):
                    e = jnp.exp((mn - dists[k]) * 200.0)
                    es.append(e)
                    tot = e if tot is None else tot + e
                r = 0.5 / tot
                for k in range(K):
                    wv = es[k] * r
                    if k == 0:
                        wv = wv + 0.5
                    acc_v[pl.ds(k * CW + base, L)] = wv

        with jax.named_scope("out_dma"):
            pltpu.sync_copy(acc_v, out_hbm.at[w])

    return sc_call


def kernel(feature1, xyz1, query_xyz, error, point_index, W, b):
    B, N, _ = xyz1.shape
    M = query_xyz.shape[1]
    K = point_index.shape[2]
    SPB = NW // B
    CW = M // SPB

    tables = xyz1.transpose(0, 2, 1).reshape(B * 3, N)
    idx = (
        point_index.astype(jnp.int32)
        .reshape(B, SPB, CW, K)
        .transpose(0, 1, 3, 2)
        .reshape(NW, K * CW)
    )
    q = (
        query_xyz.reshape(B, SPB, CW, 3)
        .transpose(0, 1, 3, 2)
        .reshape(NW, 3 * CW)
    )

    out = _make_sc_call(B, N, M, K)(tables, idx, q)
    return (
        out.reshape(B, SPB, K, CW)
        .transpose(0, 1, 3, 2)
        .reshape(B, M, K)
    )


# speedup vs baseline: 1.4216x; 1.0388x over previous
import functools

import jax
import jax.numpy as jnp
from jax import lax
from jax.experimental import pallas as pl
from jax.experimental.pallas import tpu as pltpu
from jax.experimental.pallas import tpu_sc as plsc

NC = 2
NS = 16
NW = NC * NS
L = 16

_RSQRT_MAGIC = 0x5F3759DF


def _rsqrt(x):
    i = plsc.bitcast(x, jnp.int32)
    i = _RSQRT_MAGIC - lax.shift_right_logical(i, 1)
    y = plsc.bitcast(i, jnp.float32)
    xh = x * 0.5
    for _ in range(3):
        y = y * (1.5 - xh * y * y)
    return y


def _make_sc_call(B, N, M, K):
    SPB = NW // B
    CW = M // SPB
    NQ = CW // L
    mesh = plsc.VectorSubcoreMesh(core_axis_name="c", subcore_axis_name="s")

    @functools.partial(
        pl.kernel,
        mesh=mesh,
        out_type=jax.ShapeDtypeStruct((NW, CW * K), jnp.float32),
        compiler_params=pltpu.CompilerParams(needs_layout_passes=False),
        scratch_types=[
            pltpu.VMEM((N,), jnp.float32),
            pltpu.VMEM((K * CW,), jnp.int32),
            pltpu.VMEM((K * CW,), jnp.float32),
            pltpu.VMEM((3 * CW,), jnp.float32),
            pltpu.SemaphoreType.DMA,
            pltpu.SemaphoreType.DMA,
            pltpu.SemaphoreType.DMA,
        ],
    )
    def sc_call(tab_hbm, idx_hbm, q_hbm, out_hbm,
                table_v, idx_v, acc_v, q_v, sem0, sem1, sem2):
        c = lax.axis_index("c")
        s = lax.axis_index("s")
        w = c * NS + s
        bidx = w // SPB

        with jax.named_scope("init_dma"):
            cp_i = pltpu.async_copy(idx_hbm.at[w], idx_v, sem0)
            cp_q = pltpu.async_copy(q_hbm.at[w], q_v, sem1)
            cp_t = pltpu.async_copy(tab_hbm.at[bidx * 3], table_v, sem2)
            cp_i.wait()
            cp_q.wait()
            cp_t.wait()

        for comp in range(2):
            if comp:
                with jax.named_scope(f"tab_dma{comp}"):
                    pltpu.sync_copy(tab_hbm.at[bidx * 3 + comp], table_v)

            with jax.named_scope(f"gather{comp}"):
                @plsc.parallel_loop(0, NQ, unroll=4)
                def jloop(j, comp=comp):
                    qv = q_v[pl.ds(comp * CW + j * L, L)]
                    for k in range(K):
                        off = k * CW + j * L
                        g = plsc.load_gather(table_v, [idx_v[pl.ds(off, L)]])
                        d = g - qv
                        if comp == 0:
                            acc_v[pl.ds(off, L)] = d * d
                        else:
                            acc_v[pl.ds(off, L)] = acc_v[pl.ds(off, L)] + d * d

        with jax.named_scope("tab_dma2"):
            pltpu.sync_copy(tab_hbm.at[bidx * 3 + 2], table_v)

        with jax.named_scope("gather2_softmax"):
            @plsc.parallel_loop(0, NQ, unroll=2)
            def sloop(j):
                base = j * L
                qv = q_v[pl.ds(2 * CW + base, L)]
                dists = []
                for k in range(K):
                    off = k * CW + base
                    g = plsc.load_gather(table_v, [idx_v[pl.ds(off, L)]])
                    d = g - qv
                    x = acc_v[pl.ds(off, L)] + d * d
                    x = jnp.maximum(x, 1e-30)
                    dists.append(x * _rsqrt(x))
                mn = dists[0]
                for k in range(1, K):
                    mn = jnp.minimum(mn, dists[k])
                es = []
                tot = None
                for k in range(K):
                    e = jnp.exp(mn - dists[k])
                    es.append(e)
                    tot = e if tot is None else tot + e
                r = 0.5 / tot
                for k in range(K):
                    wv = es[k] * r
                    if k == 0:
                        wv = wv + 0.5
                    acc_v[pl.ds(k * CW + base, L)] = wv

        with jax.named_scope("out_dma"):
            pltpu.sync_copy(acc_v, out_hbm.at[w])

    return sc_call


def kernel(feature1, xyz1, query_xyz, error, point_index, W, b):
    B, N, _ = xyz1.shape
    M = query_xyz.shape[1]
    K = point_index.shape[2]
    SPB = NW // B
    CW = M // SPB

    tables = (xyz1 * 200.0).transpose(0, 2, 1).reshape(B * 3, N)
    idx = (
        point_index.astype(jnp.int32)
        .reshape(B, SPB, CW, K)
        .transpose(0, 1, 3, 2)
        .reshape(NW, K * CW)
    )
    q = (
        (query_xyz * 200.0).reshape(B, SPB, CW, 3)
        .transpose(0, 1, 3, 2)
        .reshape(NW, 3 * CW)
    )

    out = _make_sc_call(B, N, M, K)(tables, idx, q)
    return (
        out.reshape(B, SPB, K, CW)
        .transpose(0, 1, 3, 2)
        .reshape(B, M, K)
    )
